# Initial kernel scaffold; baseline (speedup 1.0000x reference)
import functools

import jax
import jax.numpy as jnp
from jax import lax
from jax.experimental import pallas as pl
from jax.experimental.pallas import tpu as pltpu
from jax.experimental.pallas import tpu_sc as plsc

NC = 2
NS = 16
NW = NC * NS

D_FEAT = 128
D_EDGE = 16
D_GLOB = 16
D_OUT = D_EDGE + 2 * D_FEAT + D_GLOB

C = 80


def _edge_block(x, edge_attr, global_attr, recv_idx, send_idx):
    E = edge_attr.shape[0]
    epw = E // NW
    nchunk = epw // C

    mesh = plsc.VectorSubcoreMesh(core_axis_name="c", subcore_axis_name="s")

    @functools.partial(
        pl.kernel,
        mesh=mesh,
        out_type=jax.ShapeDtypeStruct((E, D_OUT), jnp.float32),
        scratch_types=[
            pltpu.VMEM((C,), jnp.int32),
            pltpu.VMEM((C,), jnp.int32),
            pltpu.VMEM((C, D_FEAT), jnp.float32),
            pltpu.VMEM((C, D_FEAT), jnp.float32),
            pltpu.VMEM((C, D_EDGE), jnp.float32),
            pltpu.VMEM((D_GLOB,), jnp.float32),
            pltpu.VMEM((C, D_GLOB), jnp.float32),
            pltpu.SemaphoreType.DMA,
            pltpu.SemaphoreType.DMA,
            pltpu.SemaphoreType.DMA,
        ],
    )
    def k(x_hbm, recv_hbm, send_hbm, ea_hbm, g_hbm, out_hbm,
          idx_r, idx_s, rows_r, rows_s, ea_v, g1_v, g_full,
          sem_read, sem_gather, sem_write):
        wid = lax.axis_index("s") * NC + lax.axis_index("c")
        base0 = wid * epw

        pltpu.sync_copy(g_hbm, g1_v)
        gv = g1_v[...]
        for i in range(C):
            g_full[i, :] = gv

        def chunk(i, carry):
            base = base0 + i * C
            r1 = pltpu.async_copy(recv_hbm.at[pl.ds(base, C)], idx_r, sem_read)
            r2 = pltpu.async_copy(send_hbm.at[pl.ds(base, C)], idx_s, sem_read)
            r3 = pltpu.async_copy(ea_hbm.at[pl.ds(base, C)], ea_v, sem_read)
            r1.wait()
            r2.wait()
            r3.wait()
            ga = pltpu.async_copy(x_hbm.at[idx_r], rows_r, sem_gather)
            gb = pltpu.async_copy(x_hbm.at[idx_s], rows_s, sem_gather)
            ga.wait()
            gb.wait()
            w1 = pltpu.async_copy(
                ea_v, out_hbm.at[pl.ds(base, C), pl.ds(0, D_EDGE)], sem_write)
            w2 = pltpu.async_copy(
                rows_r, out_hbm.at[pl.ds(base, C), pl.ds(D_EDGE, D_FEAT)],
                sem_write)
            w3 = pltpu.async_copy(
                rows_s,
                out_hbm.at[pl.ds(base, C), pl.ds(D_EDGE + D_FEAT, D_FEAT)],
                sem_write)
            w4 = pltpu.async_copy(
                g_full,
                out_hbm.at[pl.ds(base, C), pl.ds(D_EDGE + 2 * D_FEAT, D_GLOB)],
                sem_write)
            w1.wait()
            w2.wait()
            w3.wait()
            w4.wait()
            return carry

        lax.fori_loop(0, nchunk, chunk, 0)

    return k(x, recv_idx, send_idx, edge_attr, global_attr)


def kernel(x, edge_attr, global_attr, edge_index):
    recv_idx = edge_index[0].astype(jnp.int32)
    send_idx = edge_index[1].astype(jnp.int32)
    return _edge_block(x, edge_attr, global_attr, recv_idx, send_idx)


# speedup vs baseline: 1.6677x; 1.6677x over previous
import functools

import jax
import jax.numpy as jnp
from jax import lax
from jax.experimental import pallas as pl
from jax.experimental.pallas import tpu as pltpu
from jax.experimental.pallas import tpu_sc as plsc

NC = 2
NS = 16
NW = NC * NS

D_FEAT = 128
D_EDGE = 16
D_GLOB = 16
D_OUT = D_EDGE + 2 * D_FEAT + D_GLOB

C = 80


def _edge_block(x, edge_attr, global_attr, recv_idx, send_idx):
    E = edge_attr.shape[0]
    epw = E // NW
    nchunk = epw // C

    mesh = plsc.VectorSubcoreMesh(core_axis_name="c", subcore_axis_name="s")

    @functools.partial(
        pl.kernel,
        mesh=mesh,
        out_type=jax.ShapeDtypeStruct((E, D_OUT), jnp.float32),
        compiler_params=pltpu.CompilerParams(use_tc_tiling_on_sc=False),
        scratch_types=[
            pltpu.VMEM((C,), jnp.int32),
            pltpu.VMEM((C,), jnp.int32),
            pltpu.VMEM((C, D_FEAT), jnp.float32),
            pltpu.VMEM((C, D_FEAT), jnp.float32),
            pltpu.VMEM((C, D_EDGE), jnp.float32),
            pltpu.VMEM((D_GLOB,), jnp.float32),
            pltpu.VMEM((C, D_GLOB), jnp.float32),
            pltpu.SemaphoreType.DMA,
            pltpu.SemaphoreType.DMA,
            pltpu.SemaphoreType.DMA,
        ],
    )
    def k(x_hbm, recv_hbm, send_hbm, ea_hbm, g_hbm, out_hbm,
          idx_r, idx_s, rows_r, rows_s, ea_v, g1_v, g_full,
          sem_read, sem_gather, sem_write):
        wid = lax.axis_index("s") * NC + lax.axis_index("c")
        base0 = wid * epw

        pltpu.sync_copy(g_hbm, g1_v)
        gv = g1_v[...]
        for i in range(C):
            g_full[i, :] = gv

        def chunk(i, carry):
            base = base0 + i * C
            r1 = pltpu.async_copy(recv_hbm.at[pl.ds(base, C)], idx_r, sem_read)
            r2 = pltpu.async_copy(send_hbm.at[pl.ds(base, C)], idx_s, sem_read)
            r3 = pltpu.async_copy(ea_hbm.at[pl.ds(base, C)], ea_v, sem_read)
            r1.wait()
            r2.wait()
            r3.wait()
            ga = pltpu.async_copy(x_hbm.at[idx_r], rows_r, sem_gather)
            gb = pltpu.async_copy(x_hbm.at[idx_s], rows_s, sem_gather)
            ga.wait()
            gb.wait()
            w1 = pltpu.async_copy(
                ea_v, out_hbm.at[pl.ds(base, C), pl.ds(0, D_EDGE)], sem_write)
            w2 = pltpu.async_copy(
                rows_r, out_hbm.at[pl.ds(base, C), pl.ds(D_EDGE, D_FEAT)],
                sem_write)
            w3 = pltpu.async_copy(
                rows_s,
                out_hbm.at[pl.ds(base, C), pl.ds(D_EDGE + D_FEAT, D_FEAT)],
                sem_write)
            w4 = pltpu.async_copy(
                g_full,
                out_hbm.at[pl.ds(base, C), pl.ds(D_EDGE + 2 * D_FEAT, D_GLOB)],
                sem_write)
            w1.wait()
            w2.wait()
            w3.wait()
            w4.wait()
            return carry

        lax.fori_loop(0, nchunk, chunk, 0)

    return k(x, recv_idx, send_idx, edge_attr, global_attr)


def kernel(x, edge_attr, global_attr, edge_index):
    recv_idx = edge_index[0].astype(jnp.int32)
    send_idx = edge_index[1].astype(jnp.int32)
    return _edge_block(x, edge_attr, global_attr, recv_idx, send_idx)


# speedup vs baseline: 1.9144x; 1.1479x over previous
import functools

import jax
import jax.numpy as jnp
from jax import lax
from jax.experimental import pallas as pl
from jax.experimental.pallas import tpu as pltpu
from jax.experimental.pallas import tpu_sc as plsc

NC = 2
NS = 16
NW = NC * NS

D_FEAT = 128
D_EDGE = 16
D_GLOB = 16
D_OUT = D_EDGE + 2 * D_FEAT + D_GLOB

C = 80
NBUF = 5


def _edge_block(x, edge_attr, global_attr, recv_idx, send_idx):
    E = edge_attr.shape[0]
    epw = E // NW
    nchunk = epw // C
    nsuper = nchunk // NBUF

    mesh = plsc.VectorSubcoreMesh(core_axis_name="c", subcore_axis_name="s")

    scratch = (
        [pltpu.VMEM((C, D_FEAT), jnp.float32) for _ in range(NBUF)]
        + [pltpu.VMEM((C, D_FEAT), jnp.float32) for _ in range(NBUF)]
        + [pltpu.VMEM((C, D_EDGE), jnp.float32) for _ in range(NBUF)]
        + [
            pltpu.VMEM((epw,), jnp.int32),
            pltpu.VMEM((epw,), jnp.int32),
            pltpu.VMEM((D_GLOB,), jnp.float32),
            pltpu.VMEM((C, D_GLOB), jnp.float32),
        ]
        + [pltpu.SemaphoreType.DMA for _ in range(3 * NBUF)]
    )

    @functools.partial(
        pl.kernel,
        mesh=mesh,
        out_type=jax.ShapeDtypeStruct((E, D_OUT), jnp.float32),
        compiler_params=pltpu.CompilerParams(use_tc_tiling_on_sc=False),
        scratch_types=scratch,
    )
    def k(x_hbm, recv_hbm, send_hbm, ea_hbm, g_hbm, out_hbm, *scr):
        rows_r = scr[0:NBUF]
        rows_s = scr[NBUF:2 * NBUF]
        ea = scr[2 * NBUF:3 * NBUF]
        idxr_all, idxs_all, g1_v, g_full = scr[3 * NBUF:3 * NBUF + 4]
        sr = scr[3 * NBUF + 4:4 * NBUF + 4]
        sg = scr[4 * NBUF + 4:5 * NBUF + 4]
        sw = scr[5 * NBUF + 4:6 * NBUF + 4]

        wid = lax.axis_index("s") * NC + lax.axis_index("c")
        base0 = wid * epw

        pltpu.sync_copy(recv_hbm.at[pl.ds(base0, epw)], idxr_all)
        pltpu.sync_copy(send_hbm.at[pl.ds(base0, epw)], idxs_all)
        pltpu.sync_copy(g_hbm, g1_v)
        gv = g1_v[...]
        for i in range(C):
            g_full[i, :] = gv

        def issue_writes(base, b):
            pltpu.async_copy(
                ea[b], out_hbm.at[pl.ds(base, C), pl.ds(0, D_EDGE)], sw[b])
            pltpu.async_copy(
                rows_r[b], out_hbm.at[pl.ds(base, C), pl.ds(D_EDGE, D_FEAT)],
                sw[b])
            pltpu.async_copy(
                rows_s[b],
                out_hbm.at[pl.ds(base, C), pl.ds(D_EDGE + D_FEAT, D_FEAT)],
                sw[b])
            pltpu.async_copy(
                g_full,
                out_hbm.at[pl.ds(base, C), pl.ds(D_EDGE + 2 * D_FEAT, D_GLOB)],
                sw[b])

        def wait_writes(base, b):
            pltpu.make_async_copy(
                ea[b], out_hbm.at[pl.ds(base, C), pl.ds(0, D_EDGE)],
                sw[b]).wait()
            pltpu.make_async_copy(
                rows_r[b], out_hbm.at[pl.ds(base, C), pl.ds(D_EDGE, D_FEAT)],
                sw[b]).wait()
            pltpu.make_async_copy(
                rows_s[b],
                out_hbm.at[pl.ds(base, C), pl.ds(D_EDGE + D_FEAT, D_FEAT)],
                sw[b]).wait()
            pltpu.make_async_copy(
                g_full,
                out_hbm.at[pl.ds(base, C), pl.ds(D_EDGE + 2 * D_FEAT, D_GLOB)],
                sw[b]).wait()

        def issue_reads(j, b):
            base = base0 + j * C
            pltpu.async_copy(ea_hbm.at[pl.ds(base, C)], ea[b], sr[b])
            pltpu.async_copy(
                x_hbm.at[idxr_all.at[pl.ds(j * C, C)]], rows_r[b], sg[b])
            pltpu.async_copy(
                x_hbm.at[idxs_all.at[pl.ds(j * C, C)]], rows_s[b], sg[b])

        def wait_reads(base, b):
            pltpu.make_async_copy(
                ea_hbm.at[pl.ds(base, C)], ea[b], sr[b]).wait()
            pltpu.make_async_copy(
                x_hbm.at[pl.ds(0, C)], rows_r[b], sg[b]).wait()
            pltpu.make_async_copy(
                x_hbm.at[pl.ds(0, C)], rows_s[b], sg[b]).wait()

        def super_iter(i, carry):
            for b in range(NBUF):
                j = i * NBUF + b

                @pl.when(i > 0)
                def _():
                    wait_writes(base0 + (j - NBUF) * C, b)

                issue_reads(j, b)
            for b in range(NBUF):
                j = i * NBUF + b
                base = base0 + j * C
                wait_reads(base, b)
                issue_writes(base, b)
            return carry

        lax.fori_loop(0, nsuper, super_iter, 0)
        for b in range(NBUF):
            j = (nsuper - 1) * NBUF + b
            wait_writes(base0 + j * C, b)

    return k(x, recv_idx, send_idx, edge_attr, global_attr)


def kernel(x, edge_attr, global_attr, edge_index):
    recv_idx = edge_index[0].astype(jnp.int32)
    send_idx = edge_index[1].astype(jnp.int32)
    return _edge_block(x, edge_attr, global_attr, recv_idx, send_idx)
